# baseline (speedup 1.0000x reference)
import functools

import jax
import jax.numpy as jnp
from jax import lax
from jax.experimental import pallas as pl
from jax.experimental.pallas import tpu as pltpu

B = 128
V = 100000
NBLK = 100
CV = V // NBLK
assert CV * NBLK == V and CV % 8 == 0


def _count_body(tok_ref, hbm_ref, out_ref, win_ref, wsem, *scratch):
    bufs = scratch[:NBLK]
    sems = scratch[NBLK:]
    descs = [
        pltpu.make_async_copy(hbm_ref.at[pl.ds(j * CV, CV), :], bufs[j], sems[j])
        for j in range(NBLK)
    ]
    wdescs = []
    for b in range(B):
        t = tok_ref[b]
        d = pltpu.make_async_copy(
            hbm_ref.at[pl.ds(t, 1), :], win_ref.at[pl.ds(b, 1), :], wsem
        )
        d.start()
        wdescs.append(d)
    for d in descs:
        d.start()
    for d in wdescs:
        d.wait()
    lane = lax.broadcasted_iota(jnp.int32, (1, B), 1)
    x = jnp.zeros((1, B), jnp.float32)
    for b in range(B):
        x = x + jnp.where(lane == b, win_ref[pl.ds(b, 1), :], 0.0)
    acc = jnp.zeros((1, B), jnp.int32)
    for j in range(NBLK):
        descs[j].wait()
        blk = bufs[j][...]
        acc = acc + jnp.sum((blk > x).astype(jnp.int32), axis=0, keepdims=True)
    out_ref[...] = acc


@functools.cache
def _make_count_call():
    return pl.pallas_call(
        _count_body,
        in_specs=[
            pl.BlockSpec(memory_space=pltpu.SMEM),
            pl.BlockSpec(memory_space=pltpu.HBM),
        ],
        out_specs=pl.BlockSpec(memory_space=pltpu.VMEM),
        out_shape=jax.ShapeDtypeStruct((1, B), jnp.int32),
        scratch_shapes=[pltpu.VMEM((B, B), jnp.float32), pltpu.SemaphoreType.DMA]
        + [pltpu.VMEM((CV, B), jnp.float32) for _ in range(NBLK)]
        + [pltpu.SemaphoreType.DMA for _ in range(NBLK)],
    )


def kernel(logits, token_ids):
    tok = token_ids.astype(jnp.int32)
    counts = _make_count_call()(tok, logits.T)
    return counts.reshape(B).astype(jnp.int64)


# speedup vs baseline: 1.0586x; 1.0586x over previous
import functools

import jax
import jax.numpy as jnp
from jax import lax
from jax.experimental import pallas as pl
from jax.experimental.pallas import tpu as pltpu

B = 128
V = 100000
NBLK = 50
CV = V // NBLK
assert CV * NBLK == V and CV % 8 == 0


def _count_body(tok_ref, hbm_ref, out_ref, win_ref, wsem, *scratch):
    bufs = scratch[:NBLK]
    sems = scratch[NBLK:]
    descs = [
        pltpu.make_async_copy(hbm_ref.at[pl.ds(j * CV, CV), :], bufs[j], sems[j])
        for j in range(NBLK)
    ]
    wdescs = []
    for b in range(B):
        t = tok_ref[b]
        d = pltpu.make_async_copy(
            hbm_ref.at[pl.ds(t, 1), :], win_ref.at[pl.ds(b, 1), :], wsem
        )
        d.start()
        wdescs.append(d)
    for d in descs:
        d.start()
    for d in wdescs:
        d.wait()
    lane = lax.broadcasted_iota(jnp.int32, (1, B), 1)
    x = jnp.zeros((1, B), jnp.float32)
    for b in range(B):
        x = x + jnp.where(lane == b, win_ref[pl.ds(b, 1), :], 0.0)
    acc = jnp.zeros((1, B), jnp.int32)
    for j in range(NBLK):
        descs[j].wait()
        blk = bufs[j][...]
        acc = acc + jnp.sum((blk > x).astype(jnp.int32), axis=0, keepdims=True)
    out_ref[...] = acc


@functools.cache
def _make_count_call():
    return pl.pallas_call(
        _count_body,
        in_specs=[
            pl.BlockSpec(memory_space=pltpu.SMEM),
            pl.BlockSpec(memory_space=pltpu.HBM),
        ],
        out_specs=pl.BlockSpec(memory_space=pltpu.VMEM),
        out_shape=jax.ShapeDtypeStruct((1, B), jnp.int32),
        scratch_shapes=[pltpu.VMEM((B, B), jnp.float32), pltpu.SemaphoreType.DMA]
        + [pltpu.VMEM((CV, B), jnp.float32) for _ in range(NBLK)]
        + [pltpu.SemaphoreType.DMA for _ in range(NBLK)],
    )


def kernel(logits, token_ids):
    tok = token_ids.astype(jnp.int32)
    counts = _make_count_call()(tok, logits.T)
    return counts.reshape(B).astype(jnp.int64)
